# baseline (device time: 471993 ns/iter reference)
import os

import jax
import jax.numpy as jnp
from jax import lax
from jax.experimental import pallas as pl
from jax.experimental.pallas import tpu as pltpu

_VARIANT = os.environ.get("KERNEL_VARIANT", "full")

K = int(os.environ.get("KERNEL_K", "32"))
LK = 16
LD = 4


def kernel(x):
    m, n = x.shape
    n_out = n // 2
    half = m // 2
    ch = half // K
    lch = m // LK

    def body(
        x_ref, out_ref, xbuf, lbuf,
        xload_sem, lload_sem, lstore_sem,
        sx_send, sx_recv, sy_send, sy_recv,
    ):
        mx = lax.axis_index("x")
        my = lax.axis_index("y")
        x_peer = (1 - mx, my)
        y_peer = (mx, 1 - my)

        xsrc0 = my * half
        xdst0 = mx * m + my * half
        xin0 = (1 - mx) * m + my * half
        yin0 = (1 - mx) * m + (1 - my) * half

        if _VARIANT == "local":
            barrier = None
        else:
            barrier = pltpu.get_barrier_semaphore()
        if _VARIANT != "local":
            for peer in (x_peer, y_peer):
                pl.semaphore_signal(
                    barrier, inc=1, device_id=peer,
                    device_id_type=pl.DeviceIdType.MESH,
                )
            pl.semaphore_wait(barrier, 2)

        xloads = [None] * K
        for k in range(K if _VARIANT != "local" else 0):
            xloads[k] = pltpu.make_async_copy(
                x_ref.at[pl.ds(xsrc0 + k * ch, ch), pl.ds((1 - mx) * n_out, n_out)],
                xbuf.at[k],
                xload_sem.at[k],
            )
            xloads[k].start()

        def start_lload(j):
            c = pltpu.make_async_copy(
                x_ref.at[pl.ds(j * lch, lch), pl.ds(mx * n_out, n_out)],
                lbuf.at[j % LD],
                lload_sem.at[j % LD],
            )
            c.start()
            return c

        lloads = [None] * LK
        lstores = [None] * LK
        lstore_waited = [False] * LK
        for j in range(LD):
            lloads[j] = start_lload(j)

        sends = [None] * K
        for k in range(K if _VARIANT != "local" else 0):
            xloads[k].wait()
            sends[k] = pltpu.make_async_remote_copy(
                src_ref=xbuf.at[k],
                dst_ref=out_ref.at[pl.ds(xdst0 + k * ch, ch), :],
                send_sem=sx_send.at[k],
                recv_sem=sx_recv.at[k],
                device_id=x_peer,
                device_id_type=pl.DeviceIdType.MESH,
            )
            sends[k].start()

        def local_step(j):
            lloads[j].wait()
            lstores[j] = pltpu.make_async_copy(
                lbuf.at[j % LD],
                out_ref.at[pl.ds(mx * m + j * lch, lch), :],
                lstore_sem.at[j % LD],
            )
            lstores[j].start()
            if j + LD < LK:
                lstores[j].wait()
                lstore_waited[j] = True
                lloads[j + LD] = start_lload(j + LD)

        fwds = [None] * K
        lj = 0
        for k in range(K if _VARIANT != "local" else 0):
            recv = pltpu.make_async_remote_copy(
                src_ref=xbuf.at[k],
                dst_ref=out_ref.at[pl.ds(xin0 + k * ch, ch), :],
                send_sem=sy_send.at[k],
                recv_sem=sx_recv.at[k],
                device_id=x_peer,
                device_id_type=pl.DeviceIdType.MESH,
            )
            recv.wait_recv()
            if _VARIANT == "x":
                fwds[k] = None
                if k % 2 == 1 and lj < LK:
                    local_step(lj)
                    lj += 1
                continue
            fwds[k] = pltpu.make_async_remote_copy(
                src_ref=out_ref.at[pl.ds(xin0 + k * ch, ch), :],
                dst_ref=out_ref.at[pl.ds(xin0 + k * ch, ch), :],
                send_sem=sy_send.at[k],
                recv_sem=sy_recv.at[k],
                device_id=y_peer,
                device_id_type=pl.DeviceIdType.MESH,
            )
            fwds[k].start()
            if k % 2 == 1 and lj < LK:
                local_step(lj)
                lj += 1

        while lj < LK:
            local_step(lj)
            lj += 1

        for k in range(K if _VARIANT == "full" else 0):
            yrecv = pltpu.make_async_remote_copy(
                src_ref=xbuf.at[k],
                dst_ref=out_ref.at[pl.ds(yin0 + k * ch, ch), :],
                send_sem=sx_send.at[k],
                recv_sem=sy_recv.at[k],
                device_id=y_peer,
                device_id_type=pl.DeviceIdType.MESH,
            )
            yrecv.wait_recv()
        for k in range(K if _VARIANT != "local" else 0):
            sends[k].wait_send()
            if fwds[k] is not None:
                fwds[k].wait_send()
        for j in range(LK):
            if not lstore_waited[j]:
                lstores[j].wait()

    return pl.pallas_call(
        body,
        out_shape=jax.ShapeDtypeStruct((2 * m, n_out), x.dtype),
        in_specs=[pl.BlockSpec(memory_space=pl.ANY)],
        out_specs=pl.BlockSpec(memory_space=pl.ANY),
        scratch_shapes=[
            pltpu.VMEM((K, half // K, n_out), x.dtype),
            pltpu.VMEM((LD, m // LK, n_out), x.dtype),
            pltpu.SemaphoreType.DMA((K,)),
            pltpu.SemaphoreType.DMA((LD,)),
            pltpu.SemaphoreType.DMA((LD,)),
            pltpu.SemaphoreType.DMA((K,)),
            pltpu.SemaphoreType.DMA((K,)),
            pltpu.SemaphoreType.DMA((K,)),
            pltpu.SemaphoreType.DMA((K,)),
        ],
        compiler_params=(
            pltpu.CompilerParams(
                collective_id=0, vmem_limit_bytes=56 * 1024 * 1024
            )
            if _VARIANT != "local"
            else pltpu.CompilerParams(vmem_limit_bytes=56 * 1024 * 1024)
        ),
    )(x)


# device time: 454930 ns/iter; 1.0375x vs baseline; 1.0375x over previous
import os

import jax
import jax.numpy as jnp
from jax import lax
from jax.experimental import pallas as pl
from jax.experimental.pallas import tpu as pltpu

_VARIANT = os.environ.get("KERNEL_VARIANT", "full")

K = int(os.environ.get("KERNEL_K", "32"))
LK = 16
LD = 4


def kernel(x):
    m, n = x.shape
    n_out = n // 2
    half = m // 2
    ch = half // K
    lch = m // LK

    def body(
        x_ref, out_ref, xbuf, lbuf,
        xload_sem, lload_sem, lstore_sem,
        sx_send, sx_recv, sy_send, sy_recv,
    ):
        mx = lax.axis_index("x")
        my = lax.axis_index("y")
        x_peer = (1 - mx, my)
        y_peer = (mx, 1 - my)

        xsrc0 = my * half
        xdst0 = mx * m + my * half
        xin0 = (1 - mx) * m + my * half
        yin0 = (1 - mx) * m + (1 - my) * half

        if _VARIANT == "local":
            barrier = None
        else:
            barrier = pltpu.get_barrier_semaphore()
        if _VARIANT != "local":
            for peer in (x_peer, y_peer):
                pl.semaphore_signal(
                    barrier, inc=1, device_id=peer,
                    device_id_type=pl.DeviceIdType.MESH,
                )
            pl.semaphore_wait(barrier, 2)

        xloads = [None] * K
        for k in range(K if _VARIANT != "local" else 0):
            xloads[k] = pltpu.make_async_copy(
                x_ref.at[pl.ds(xsrc0 + k * ch, ch), pl.ds((1 - mx) * n_out, n_out)],
                xbuf.at[k],
                xload_sem.at[k],
            )
            xloads[k].start()

        def start_lload(j):
            c = pltpu.make_async_copy(
                x_ref.at[pl.ds(j * lch, lch), pl.ds(mx * n_out, n_out)],
                lbuf.at[j % LD],
                lload_sem.at[j % LD],
            )
            c.start()
            return c

        lloads = [None] * LK
        lstores = [None] * LK
        lstore_waited = [False] * LK
        for j in range(LD):
            lloads[j] = start_lload(j)

        sends = [None] * K
        for k in range(K if _VARIANT != "local" else 0):
            xloads[k].wait()
            sends[k] = pltpu.make_async_remote_copy(
                src_ref=xbuf.at[k],
                dst_ref=(
                    xbuf.at[k] if _VARIANT == "xv"
                    else out_ref.at[pl.ds(xdst0 + k * ch, ch), :]
                ),
                send_sem=sx_send.at[k],
                recv_sem=sx_recv.at[k],
                device_id=x_peer,
                device_id_type=pl.DeviceIdType.MESH,
            )
            sends[k].start()

        def local_step(j):
            lloads[j].wait()
            lstores[j] = pltpu.make_async_copy(
                lbuf.at[j % LD],
                out_ref.at[pl.ds(mx * m + j * lch, lch), :],
                lstore_sem.at[j % LD],
            )
            lstores[j].start()
            if j + LD < LK:
                lstores[j].wait()
                lstore_waited[j] = True
                lloads[j + LD] = start_lload(j + LD)

        fwds = [None] * K
        lj = 0
        for k in range(K if _VARIANT != "local" else 0):
            recv = pltpu.make_async_remote_copy(
                src_ref=xbuf.at[k],
                dst_ref=(
                    xbuf.at[k] if _VARIANT == "xv"
                    else out_ref.at[pl.ds(xin0 + k * ch, ch), :]
                ),
                send_sem=sy_send.at[k],
                recv_sem=sx_recv.at[k],
                device_id=x_peer,
                device_id_type=pl.DeviceIdType.MESH,
            )
            recv.wait_recv()
            if _VARIANT in ("x", "xv"):
                fwds[k] = None
                if k % 2 == 1 and lj < LK:
                    local_step(lj)
                    lj += 1
                continue
            fwds[k] = pltpu.make_async_remote_copy(
                src_ref=out_ref.at[pl.ds(xin0 + k * ch, ch), :],
                dst_ref=out_ref.at[pl.ds(xin0 + k * ch, ch), :],
                send_sem=sy_send.at[k],
                recv_sem=sy_recv.at[k],
                device_id=y_peer,
                device_id_type=pl.DeviceIdType.MESH,
            )
            fwds[k].start()
            if k % 2 == 1 and lj < LK:
                local_step(lj)
                lj += 1

        while lj < LK:
            local_step(lj)
            lj += 1

        for k in range(K if _VARIANT == "full" else 0):
            yrecv = pltpu.make_async_remote_copy(
                src_ref=xbuf.at[k],
                dst_ref=out_ref.at[pl.ds(yin0 + k * ch, ch), :],
                send_sem=sx_send.at[k],
                recv_sem=sy_recv.at[k],
                device_id=y_peer,
                device_id_type=pl.DeviceIdType.MESH,
            )
            yrecv.wait_recv()
        for k in range(K if _VARIANT != "local" else 0):
            sends[k].wait_send()
            if fwds[k] is not None:
                fwds[k].wait_send()
        for j in range(LK):
            if not lstore_waited[j]:
                lstores[j].wait()

    return pl.pallas_call(
        body,
        out_shape=jax.ShapeDtypeStruct((2 * m, n_out), x.dtype),
        in_specs=[pl.BlockSpec(memory_space=pl.ANY)],
        out_specs=pl.BlockSpec(memory_space=pl.ANY),
        scratch_shapes=[
            pltpu.VMEM((K, half // K, n_out), x.dtype),
            pltpu.VMEM((LD, m // LK, n_out), x.dtype),
            pltpu.SemaphoreType.DMA((K,)),
            pltpu.SemaphoreType.DMA((LD,)),
            pltpu.SemaphoreType.DMA((LD,)),
            pltpu.SemaphoreType.DMA((K,)),
            pltpu.SemaphoreType.DMA((K,)),
            pltpu.SemaphoreType.DMA((K,)),
            pltpu.SemaphoreType.DMA((K,)),
        ],
        compiler_params=(
            pltpu.CompilerParams(
                collective_id=0, vmem_limit_bytes=56 * 1024 * 1024
            )
            if _VARIANT != "local"
            else pltpu.CompilerParams(vmem_limit_bytes=56 * 1024 * 1024)
        ),
    )(x)
